# Initial kernel scaffold; baseline (speedup 1.0000x reference)
import jax
import jax.numpy as jnp
from jax.experimental import pallas as pl


def kernel(input_1, tensor_index, W1, b1, W2, b2, W3, b3, W4, b4):
    raise NotImplementedError("write your pallas kernel here")


# speedup vs baseline: 7.1635x; 7.1635x over previous
import dataclasses
import functools

import jax
import jax.numpy as jnp
from jax import lax
from jax.experimental import pallas as pl
from jax.experimental.pallas import tpu as pltpu
from jax.experimental.pallas import tpu_sc as plsc

K = 1000000
H_GRID = 200
W_GRID = 70400

NW = 32
NB = 128
BWCOL = 550
GW = 560
GRID_CELLS = H_GRID * GW
DUMP = GRID_CELLS
GRID_TOT = GRID_CELLS + 16
CAP = 464
NEG = -9999999.0

VPW = 62500 // NW
VEXTRA = 62500 - VPW * NW
CHUNK_V = 128
NCHUNK = (VPW + 1 + CHUNK_V - 1) // CHUNK_V
PADK = 16 * 62500 + CHUNK_V * 16

DIV_M = 3814
DIV_S = 21


def _mlp_body(h_ref, w1_ref, b1_ref, w2_ref, b2_ref, w3_ref, b3_ref,
              w4_ref, b4_ref, o_ref):
    a = h_ref[...]
    z = jnp.maximum(jnp.dot(w1_ref[...], a,
                            preferred_element_type=jnp.float32) + b1_ref[...], 0.0)
    z = jnp.maximum(jnp.dot(w2_ref[...], z,
                            preferred_element_type=jnp.float32) + b2_ref[...], 0.0)
    z = jnp.maximum(jnp.dot(w3_ref[...], z,
                            preferred_element_type=jnp.float32) + b3_ref[...], 0.0)
    o_ref[...] = jnp.dot(w4_ref[...], z,
                         preferred_element_type=jnp.float32) + b4_ref[...]


def _mlp(h2d, W1, b1, W2, b2, W3, b3, W4, b4):
    BK = 8192
    grid = (pl.cdiv(K, BK),)
    full = lambda shp: pl.BlockSpec(shp, lambda i: (0, 0))
    return pl.pallas_call(
        _mlp_body,
        grid=grid,
        in_specs=[
            pl.BlockSpec((4, BK), lambda i: (0, i)),
            full((18, 4)), full((18, 1)),
            full((36, 18)), full((36, 1)),
            full((36, 36)), full((36, 1)),
            full((1, 36)), full((1, 1)),
        ],
        out_specs=pl.BlockSpec((1, BK), lambda i: (0, i)),
        out_shape=jax.ShapeDtypeStruct((1, K), jnp.float32),
    )(h2d, W1, b1[:, None], W2, b2[:, None], W3, b3[:, None], W4, b4[:, None])


def _route_kernel(h_hbm, w_hbm, x_hbm, cells_hbm, xs_hbm, counts_hbm,
                  stage_c, stage_x, counts, hbuf, wbuf, xbuf):
    wid = lax.axis_index("s") * 2 + lax.axis_index("c")
    v0 = wid * VPW + jnp.minimum(wid, VEXTRA)
    nv = VPW + jnp.where(wid < VEXTRA, 1, 0)

    zero16 = jnp.zeros((16,), jnp.int32)
    for j in range(0, NB, 16):
        counts[pl.ds(j, 16)] = zero16

    for c in range(NCHUNK):
        off = (v0 + c * CHUNK_V) * 16
        pltpu.sync_copy(h_hbm.at[pl.ds(off, CHUNK_V * 16)], hbuf)
        pltpu.sync_copy(w_hbm.at[pl.ds(off, CHUNK_V * 16)], wbuf)
        pltpu.sync_copy(x_hbm.at[pl.ds(off, CHUNK_V * 16)], xbuf)
        nproc = jnp.clip(nv - c * CHUNK_V, 0, CHUNK_V)

        def body(v, carry):
            hh = hbuf[pl.ds(v * 16, 16)]
            ww = wbuf[pl.ds(v * 16, 16)]
            xx = xbuf[pl.ds(v * 16, 16)]
            q = lax.shift_right_logical(ww * DIV_M, DIV_S)
            wl = ww - q * BWCOL
            fix = wl < 0
            b = jnp.where(fix, q - 1, q)
            wl = jnp.where(fix, wl + BWCOL, wl)
            cell = hh * GW + wl
            cnt, last = plsc.scan_count(b)
            base = plsc.load_gather(counts, [b])
            slot = base + cnt - 1
            addr = b * CAP + slot
            plsc.store_scatter(stage_c, [addr], cell)
            plsc.store_scatter(stage_x, [addr], xx)
            plsc.store_scatter(counts, [b], base + cnt, mask=last)
            return carry

        lax.fori_loop(0, nproc, body, 0)

    iota = lax.iota(jnp.int32, 16)
    dump16 = jnp.full((16,), DUMP, jnp.int32)
    neg16 = jnp.full((16,), NEG, jnp.float32)

    for g in range(NB // 16):
        n16 = counts[pl.ds(g * 16, 16)]
        pad16 = (-n16) & 15
        bvec = g * 16 + iota
        base_addr = bvec * CAP + n16
        for j in range(15):
            mask = j < pad16
            plsc.store_scatter(stage_c, [base_addr + j], dump16, mask=mask)
            plsc.store_scatter(stage_x, [base_addr + j], neg16, mask=mask)
        counts[pl.ds(g * 16, 16)] = n16 + pad16

    pltpu.sync_copy(stage_c, cells_hbm.at[pl.ds(wid * (NB * CAP), NB * CAP)])
    pltpu.sync_copy(stage_x, xs_hbm.at[pl.ds(wid * (NB * CAP), NB * CAP)])
    pltpu.sync_copy(counts, counts_hbm.at[pl.ds(wid * NB, NB)])


def _reduce_kernel(cells_hbm, xs_hbm, counts_t_hbm, out_hbm,
                   grid_v, cbuf, xbuf, cnts, col):
    wid = lax.axis_index("s") * 2 + lax.axis_index("c")
    neg16 = jnp.full((16,), NEG, jnp.float32)

    def ig(j, carry):
        grid_v[pl.ds(j * 16, 16)] = neg16
        return carry

    lax.fori_loop(0, GRID_TOT // 16, ig, 0)

    for k in range(4):
        b = wid * 4 + k
        pltpu.sync_copy(counts_t_hbm.at[pl.ds(b * NW, NW)], cnts)
        n_lo = cnts[pl.ds(0, 16)]
        n_hi = cnts[pl.ds(16, 16)]

        def vec(v, c2, cb=None, xb=None):
            cell = cb[pl.ds(v * 16, 16)]
            xv = xb[pl.ds(v * 16, 16)]
            cnt, last = plsc.scan_count(cell)
            plsc.store_scatter(grid_v, [cell], xv, mask=last)
            return c2

        for w in range(NW):
            pltpu.sync_copy(
                cells_hbm.at[pl.ds((w * NB + b) * CAP, CAP)], cbuf)
            pltpu.sync_copy(
                xs_hbm.at[pl.ds((w * NB + b) * CAP, CAP)], xbuf)
            n = (n_lo if w < 16 else n_hi)[w % 16]
            lax.fori_loop(0, lax.shift_right_logical(n, 4),
                          functools.partial(vec, cb=cbuf, xb=xbuf), 0)

        for j in range(GW // 16):
            def hred(hh, acc):
                a = pl.ds(hh * GW + j * 16, 16)
                g = grid_v[a]
                grid_v[a] = neg16
                return jnp.maximum(acc, g)

            acc = lax.fori_loop(0, H_GRID, hred, neg16)
            col[pl.ds(j * 16, 16)] = acc

        pltpu.sync_copy(col, out_hbm.at[pl.ds(b * GW, GW)])


def kernel(input_1, tensor_index, W1, b1, W2, b2, W3, b3, W4, b4):
    h2d = input_1.reshape(4, K)
    x = _mlp(h2d, W1, b1, W2, b2, W3, b3, W4, b4).reshape(K)

    h_arr = tensor_index[:, 0].astype(jnp.int32)
    w_arr = tensor_index[:, 1].astype(jnp.int32)
    pad = PADK - K
    h_p = jnp.pad(h_arr, (0, pad))
    w_p = jnp.pad(w_arr, (0, pad))
    x_p = jnp.pad(x, (0, pad))

    mesh = plsc.VectorSubcoreMesh(core_axis_name="c", subcore_axis_name="s")
    cp = pltpu.CompilerParams()
    if "needs_layout_passes" in pltpu.CompilerParams.__dataclass_fields__:
        cp = dataclasses.replace(cp, needs_layout_passes=False)

    route = pl.kernel(
        _route_kernel,
        mesh=mesh,
        compiler_params=cp,
        out_type=[
            jax.ShapeDtypeStruct((NW * NB * CAP,), jnp.int32),
            jax.ShapeDtypeStruct((NW * NB * CAP,), jnp.float32),
            jax.ShapeDtypeStruct((NW * NB,), jnp.int32),
        ],
        scratch_types=[
            pltpu.VMEM((NB * CAP,), jnp.int32),
            pltpu.VMEM((NB * CAP,), jnp.float32),
            pltpu.VMEM((NB,), jnp.int32),
            pltpu.VMEM((CHUNK_V * 16,), jnp.int32),
            pltpu.VMEM((CHUNK_V * 16,), jnp.int32),
            pltpu.VMEM((CHUNK_V * 16,), jnp.float32),
        ],
    )
    cells, xs, counts = route(h_p, w_p, x_p)
    counts_t = counts.reshape(NW, NB).T.reshape(NW * NB)

    reduce_k = pl.kernel(
        _reduce_kernel,
        mesh=mesh,
        compiler_params=cp,
        out_type=jax.ShapeDtypeStruct((NB * GW,), jnp.float32),
        scratch_types=[
            pltpu.VMEM((GRID_TOT,), jnp.float32),
            pltpu.VMEM((CAP,), jnp.int32),
            pltpu.VMEM((CAP,), jnp.float32),
            pltpu.VMEM((NW,), jnp.int32),
            pltpu.VMEM((GW,), jnp.float32),
        ],
    )
    outp = reduce_k(cells, xs, counts_t)

    out = outp.reshape(NB, GW)[:, :BWCOL].reshape(1, W_GRID, 1)
    return (out, 1)
